# baseline (speedup 1.0000x reference)
import functools

import numpy as np
import jax
import jax.numpy as jnp
from jax import lax
from jax.experimental import pallas as pl
from jax.experimental.pallas import tpu as pltpu
from jax.experimental.pallas import tpu_sc as plsc

H = W = 4096
NW = 32
CPW = 64
RPC = 2
BPC = RPC * W
SUB = 2048
LOCCAP = 16384


@functools.lru_cache(maxsize=1)
def _tables():
    def _bin_map():
        cx, cy = W // 2, H // 2
        max_r = np.sqrt(cx**2 + cy**2)
        y = jnp.arange(H, dtype=jnp.float32)[:, None] - cy
        x = jnp.arange(W, dtype=jnp.float32)[None, :] - cx
        r = jnp.sqrt(x**2 + y**2)
        theta = jnp.arctan2(y, x)
        r_idx = jnp.round(r * (H - 1) / max_r).astype(jnp.int32)
        t_idx = jnp.round(jnp.mod(theta + 2.0 * np.pi, 2.0 * np.pi)
                          * (W - 1) / (2.0 * np.pi)).astype(jnp.int32)
        t_bin = jnp.clip(t_idx, 0, H - 1)
        r_bin = jnp.clip(r_idx, 0, W - 1)
        return t_bin * W + r_bin

    try:
        flat = np.asarray(jax.jit(_bin_map)()).astype(np.int64).ravel()
    except Exception:
        cx, cy = W // 2, H // 2
        max_r = np.float32(np.sqrt(cx**2 + cy**2))
        two_pi = np.float32(2.0 * np.pi)
        y = np.arange(H, dtype=np.float32)[:, None] - np.float32(cy)
        x = np.arange(W, dtype=np.float32)[None, :] - np.float32(cx)
        r = np.sqrt(x**2 + y**2, dtype=np.float32)
        theta = np.arctan2(y, x, dtype=np.float32)
        r_idx = np.round((r * np.float32(H - 1)) / max_r).astype(np.int32)
        tm = np.fmod(theta + two_pi, two_pi)
        t_idx = np.round((tm * np.float32(W - 1)) / two_pi).astype(np.int32)
        flat = (np.clip(t_idx, 0, H - 1).astype(np.int64) * W
                + np.clip(r_idx, 0, W - 1)).ravel()
    order = np.argsort(flat, kind="stable").astype(np.int32)
    counts = np.bincount(flat, minlength=H * W).astype(np.int64)
    starts = np.zeros(H * W + 1, np.int64)
    np.cumsum(counts, out=starts[1:])
    packed = ((starts[:-1] << 3) | counts).astype(np.int32)

    nchunks = NW * CPW
    e0 = starts[np.arange(nchunks) * BPC]
    e1 = starts[(np.arange(nchunks) + 1) * BPC]
    e0a = e0 & ~np.int64(SUB - 1)
    nsub = -(-(e1 - e0a) // SUB)
    assert int((e1 - e0a).max()) <= LOCCAP

    el = (e1 - e0).astype(np.int64)
    cum = np.concatenate([[0], np.cumsum(el)])
    cuts = [0]
    for wkr in range(1, NW):
        ideal = cum[cuts[-1]] + (cum[-1] - cum[cuts[-1]]) / (NW - wkr + 1)
        j = int(np.searchsorted(cum, ideal))
        if (j - cuts[-1]) % 2:
            j += 1
        j = min(j, nchunks - 2 * (NW - wkr))
        j = max(j, cuts[-1] + 2)
        cuts.append(j)
    cuts.append(nchunks)
    assert len(cuts) == NW + 1
    ncw = np.diff(cuts)
    assert ncw.max() <= 94 and ncw.min() >= 2 and (ncw % 2 == 0).all()

    tbl = np.zeros((NW, 256), np.int32)
    for wkr in range(NW):
        lo, hi = cuts[wkr], cuts[wkr + 1]
        n = hi - lo
        tbl[wkr, :n] = e0a[lo:hi]
        tbl[wkr, n:96] = e0a[hi - 1]
        tbl[wkr, 96:96 + n] = nsub[lo:hi]
        tbl[wkr, 96 + n:192] = nsub[hi - 1]
        tbl[wkr, 192] = lo
        tbl[wkr, 208] = n

    order_pad = np.pad(order, (0, LOCCAP))
    return order_pad, packed, tbl


def _hist_body(img_hbm, order_hbm, packed_hbm, tbl_hbm, out_hbm,
               tblv, idxv0, idxv1, gv0, gv1, pkv0, pkv1, outv,
               semo0, semo1, semg0, semg1):
    w = lax.axis_index("s") * 2 + lax.axis_index("c")
    pltpu.sync_copy(tbl_hbm.at[w], tblv)
    cstart = tblv[pl.ds(192, 16)][0]
    ncw = tblv[pl.ds(208, 16)][0]

    def meta(c):
        lane = lax.bitwise_and(c, 15)
        base = pl.multiple_of(lax.bitwise_and(c, ~15), 16)
        ve = tblv[pl.ds(base, 16)]
        vn = tblv[pl.ds(base + 96, 16)]
        e0a = jnp.int32(0)
        nsub = jnp.int32(0)
        for k in range(16):
            sel = lane == k
            e0a = jnp.where(sel, ve[k], e0a)
            nsub = jnp.where(sel, vn[k], nsub)
        return pl.multiple_of(e0a, SUB), nsub

    def j_of(c):
        return pl.multiple_of((cstart + c) * BPC, BPC)

    def order_copies(e0a, s):
        so = pl.multiple_of(s * SUB, SUB)
        src = order_hbm.at[pl.ds(pl.multiple_of(e0a + so, SUB), SUB)]
        return src, so

    def fire_order_idx(e0a, nsub, idxb, semo):
        def ld(s, cc):
            src, so = order_copies(e0a, s)
            pltpu.async_copy(src, idxb.at[pl.ds(so, SUB)], semo)
            return cc
        lax.fori_loop(0, nsub, ld, 0)

    def fire_packed(c, pkb, semo):
        pltpu.async_copy(packed_hbm.at[pl.ds(j_of(c), BPC)], pkb, semo)

    def drain_order(c, e0a, nsub, idxb, pkb, semo):
        def ld(s, cc):
            src, so = order_copies(e0a, s)
            pltpu.make_async_copy(src, idxb.at[pl.ds(so, SUB)], semo).wait()
            return cc
        lax.fori_loop(0, nsub, ld, 0)
        pltpu.make_async_copy(packed_hbm.at[pl.ds(j_of(c), BPC)], pkb,
                              semo).wait()

    def fire_gath(nsub, idxb, gb, semg):
        def fire(s, cc):
            so = pl.multiple_of(s * SUB, SUB)
            pltpu.async_copy(img_hbm.at[idxb.at[pl.ds(so, SUB)]],
                             gb.at[pl.ds(so, SUB)], semg)
            return cc
        lax.fori_loop(0, nsub, fire, 0)

    def drain_gath(nsub, idxb, gb, semg):
        def drain(s, cc):
            so = pl.multiple_of(s * SUB, SUB)
            pltpu.make_async_copy(img_hbm.at[idxb.at[pl.ds(so, SUB)]],
                                  gb.at[pl.ds(so, SUB)], semg).wait()
            return cc
        lax.fori_loop(0, nsub, drain, 0)

    def compute(c, e0a, pkb, gb):
        def vec_body(v, cc):
            vo = pl.multiple_of(v * 16, 16)
            pk = pkb[pl.ds(vo, 16)]
            st = lax.shift_right_logical(pk, 3) - e0a
            cnt = lax.bitwise_and(pk, 7)
            acc = jnp.zeros((16,), jnp.float32)
            for k in range(4):
                m = cnt > k
                val = plsc.load_gather(gb, [st + k])
                acc = acc + jnp.where(m, val, 0.0)
            inv = jnp.where(
                cnt == 1, np.float32(1.0),
                jnp.where(cnt == 2, np.float32(0.5),
                          jnp.where(cnt == 3, np.float32(1.0 / 3.0),
                                    jnp.where(cnt == 4, np.float32(0.25),
                                              np.float32(0.0)))))
            outv[pl.ds(vo, 16)] = acc * inv
            return cc

        lax.fori_loop(0, BPC // 16, vec_body, 0)
        pltpu.sync_copy(outv, out_hbm.at[pl.ds(j_of(c), BPC)])

    bufs = ((idxv0, gv0, pkv0, semo0, semg0),
            (idxv1, gv1, pkv1, semo1, semg1))

    e0a0, ns0 = meta(jnp.int32(0))
    fire_order_idx(e0a0, ns0, idxv0, semo0)
    fire_packed(jnp.int32(0), pkv0, semo0)
    drain_order(jnp.int32(0), e0a0, ns0, idxv0, pkv0, semo0)
    fire_gath(ns0, idxv0, gv0, semg0)
    e0a1, ns1 = meta(jnp.int32(1))
    fire_order_idx(e0a1, ns1, idxv1, semo1)
    fire_packed(jnp.int32(1), pkv1, semo1)

    def group_body(g, carry):
        for poff in (0, 1):
            c = g * 2 + poff
            idxc, gc, pkc, semoc, semgc = bufs[poff]
            idxn, gn, pkn, semon, semgn = bufs[1 - poff]
            e0a_c, ns_c = meta(c)
            e0a_n, ns_n = meta(c + 1)
            e0a_f, ns_f = meta(c + 2)

            @pl.when(c < ncw - 1)
            def _():
                drain_order(c + 1, e0a_n, ns_n, idxn, pkn, semon)
                fire_gath(ns_n, idxn, gn, semgn)

            drain_gath(ns_c, idxc, gc, semgc)

            @pl.when(c < ncw - 2)
            def _():
                fire_order_idx(e0a_f, ns_f, idxc, semoc)

            compute(c, e0a_c, pkc, gc)

            @pl.when(c < ncw - 2)
            def _():
                fire_packed(c + 2, pkc, semoc)
        return carry

    lax.fori_loop(0, lax.shift_right_logical(ncw, 1), group_body, 0)


_mesh = plsc.VectorSubcoreMesh(core_axis_name="c", subcore_axis_name="s")

_hist = pl.kernel(
    _hist_body, mesh=_mesh,
    compiler_params=pltpu.CompilerParams(needs_layout_passes=False),
    out_type=jax.ShapeDtypeStruct((H * W,), jnp.float32),
    scratch_types=[
        pltpu.VMEM((256,), jnp.int32),
        pltpu.VMEM((LOCCAP,), jnp.int32),
        pltpu.VMEM((LOCCAP,), jnp.int32),
        pltpu.VMEM((LOCCAP,), jnp.float32),
        pltpu.VMEM((LOCCAP,), jnp.float32),
        pltpu.VMEM((BPC,), jnp.int32),
        pltpu.VMEM((BPC,), jnp.int32),
        pltpu.VMEM((BPC,), jnp.float32),
        pltpu.SemaphoreType.DMA,
        pltpu.SemaphoreType.DMA,
        pltpu.SemaphoreType.DMA,
        pltpu.SemaphoreType.DMA,
    ])


def kernel(img):
    order_pad, packed, tbl = _tables()
    P = _hist(img.reshape(-1), order_pad, packed, tbl).reshape(H, W)
    z = img[0, 0]
    col = jnp.arange(W - 1, dtype=jnp.int32)

    def _fix(p):
        c = p[:, : W - 1]
        nz = jnp.any(c != 0.0, axis=0)
        last = jnp.max(jnp.where(nz, col, -1))
        return jnp.where(col[None, :] < last, c, 0.0)

    return lax.cond(z != 0.0, lambda p: p[:, : W - 1], _fix, P)


try:
    _tables()
except Exception:
    pass


# speedup vs baseline: 1.1019x; 1.1019x over previous
import functools

import numpy as np
import jax
import jax.numpy as jnp
from jax import lax
from jax.experimental import pallas as pl
from jax.experimental.pallas import tpu as pltpu
from jax.experimental.pallas import tpu_sc as plsc

H = W = 4096
NW = 32
CPW = 64
RPC = 2
BPC = RPC * W
SUB = 2048
LOCCAP = 16384


@functools.lru_cache(maxsize=1)
def _tables():
    def _bin_map():
        cx, cy = W // 2, H // 2
        max_r = np.sqrt(cx**2 + cy**2)
        y = jnp.arange(H, dtype=jnp.float32)[:, None] - cy
        x = jnp.arange(W, dtype=jnp.float32)[None, :] - cx
        r = jnp.sqrt(x**2 + y**2)
        theta = jnp.arctan2(y, x)
        r_idx = jnp.round(r * (H - 1) / max_r).astype(jnp.int32)
        t_idx = jnp.round(jnp.mod(theta + 2.0 * np.pi, 2.0 * np.pi)
                          * (W - 1) / (2.0 * np.pi)).astype(jnp.int32)
        t_bin = jnp.clip(t_idx, 0, H - 1)
        r_bin = jnp.clip(r_idx, 0, W - 1)
        return t_bin * W + r_bin

    try:
        flat = np.asarray(jax.jit(_bin_map)()).astype(np.int64).ravel()
    except Exception:
        cx, cy = W // 2, H // 2
        max_r = np.float32(np.sqrt(cx**2 + cy**2))
        two_pi = np.float32(2.0 * np.pi)
        y = np.arange(H, dtype=np.float32)[:, None] - np.float32(cy)
        x = np.arange(W, dtype=np.float32)[None, :] - np.float32(cx)
        r = np.sqrt(x**2 + y**2, dtype=np.float32)
        theta = np.arctan2(y, x, dtype=np.float32)
        r_idx = np.round((r * np.float32(H - 1)) / max_r).astype(np.int32)
        tm = np.fmod(theta + two_pi, two_pi)
        t_idx = np.round((tm * np.float32(W - 1)) / two_pi).astype(np.int32)
        flat = (np.clip(t_idx, 0, H - 1).astype(np.int64) * W
                + np.clip(r_idx, 0, W - 1)).ravel()
    order = np.argsort(flat, kind="stable").astype(np.int32)
    counts = np.bincount(flat, minlength=H * W).astype(np.int64)
    starts = np.zeros(H * W + 1, np.int64)
    np.cumsum(counts, out=starts[1:])
    packed = None

    nchunks = NW * CPW
    e0 = starts[np.arange(nchunks) * BPC]
    e1 = starts[(np.arange(nchunks) + 1) * BPC]
    e0a = e0 & ~np.int64(SUB - 1)
    nsub = -(-(e1 - e0a) // SUB)
    assert int((e1 - e0a).max()) <= LOCCAP
    packed = (((starts[:-1] - np.repeat(e0a, BPC)) << 3)
              | counts).astype(np.int32)
    assert int((starts[:-1] - np.repeat(e0a, BPC)).max()) < LOCCAP
    assert int((starts[:-1] - np.repeat(e0a, BPC)).min()) >= 0

    cuts = list(range(0, nchunks + 1, CPW))
    assert len(cuts) == NW + 1
    ncw = np.diff(cuts)
    assert ncw.max() <= 94 and ncw.min() >= 2 and (ncw % 2 == 0).all()

    tbl = np.zeros((NW, 256), np.int32)
    for wkr in range(NW):
        lo, hi = cuts[wkr], cuts[wkr + 1]
        n = hi - lo
        tbl[wkr, :n] = e0a[lo:hi]
        tbl[wkr, n:96] = e0a[hi - 1]
        tbl[wkr, 96:96 + n] = nsub[lo:hi]
        tbl[wkr, 96 + n:192] = nsub[hi - 1]
        tbl[wkr, 192] = lo
        tbl[wkr, 208] = n

    order_pad = np.pad(order, (0, LOCCAP))
    return order_pad, packed, tbl


def _hist_body(img_hbm, order_hbm, packed_hbm, tbl_hbm, out_hbm,
               tblv, idxv0, idxv1, gv0, gv1, pkv0, pkv1, outv,
               semo0, semo1, semg0, semg1):
    w = lax.axis_index("s") * 2 + lax.axis_index("c")
    pltpu.sync_copy(tbl_hbm.at[w], tblv)
    cstart = tblv[pl.ds(192, 16)][0]
    ncw = tblv[pl.ds(208, 16)][0]

    def meta(c):
        lane = lax.bitwise_and(c, 15)
        base = pl.multiple_of(lax.bitwise_and(c, ~15), 16)
        ve = tblv[pl.ds(base, 16)]
        vn = tblv[pl.ds(base + 96, 16)]
        e0a = jnp.int32(0)
        nsub = jnp.int32(0)
        for k in range(16):
            sel = lane == k
            e0a = jnp.where(sel, ve[k], e0a)
            nsub = jnp.where(sel, vn[k], nsub)
        return pl.multiple_of(e0a, SUB), nsub

    def j_of(c):
        return pl.multiple_of((cstart + c) * BPC, BPC)

    def order_copies(e0a, s):
        so = pl.multiple_of(s * SUB, SUB)
        src = order_hbm.at[pl.ds(pl.multiple_of(e0a + so, SUB), SUB)]
        return src, so

    def fire_order_idx(e0a, nsub, idxb, semo):
        def ld(s, cc):
            src, so = order_copies(e0a, s)
            pltpu.async_copy(src, idxb.at[pl.ds(so, SUB)], semo)
            return cc
        lax.fori_loop(0, nsub, ld, 0)

    def fire_packed(c, pkb, semo):
        pltpu.async_copy(packed_hbm.at[pl.ds(j_of(c), BPC)], pkb, semo)

    def drain_order(c, e0a, nsub, idxb, pkb, semo):
        def ld(s, cc):
            src, so = order_copies(e0a, s)
            pltpu.make_async_copy(src, idxb.at[pl.ds(so, SUB)], semo).wait()
            return cc
        lax.fori_loop(0, nsub, ld, 0)
        pltpu.make_async_copy(packed_hbm.at[pl.ds(j_of(c), BPC)], pkb,
                              semo).wait()

    def fire_gath(nsub, idxb, gb, semg):
        def fire(s, cc):
            so = pl.multiple_of(s * SUB, SUB)
            pltpu.async_copy(img_hbm.at[idxb.at[pl.ds(so, SUB)]],
                             gb.at[pl.ds(so, SUB)], semg)
            return cc
        lax.fori_loop(0, nsub, fire, 0)

    def drain_gath(nsub, idxb, gb, semg):
        def drain(s, cc):
            so = pl.multiple_of(s * SUB, SUB)
            pltpu.make_async_copy(img_hbm.at[idxb.at[pl.ds(so, SUB)]],
                                  gb.at[pl.ds(so, SUB)], semg).wait()
            return cc
        lax.fori_loop(0, nsub, drain, 0)

    def compute(c, pkb, gb):
        @plsc.parallel_loop(0, BPC // 16, unroll=4)
        def vec_body(v):
            vo = pl.multiple_of(v * 16, 16)
            pk = pkb[pl.ds(vo, 16)]
            st = lax.shift_right_logical(pk, 3)
            cnt = lax.bitwise_and(pk, 7)
            acc = jnp.zeros((16,), jnp.float32)
            for k in range(4):
                m = cnt > k
                val = plsc.load_gather(gb, [st + k])
                acc = acc + jnp.where(m, val, 0.0)
            inv = jnp.where(
                cnt == 1, np.float32(1.0),
                jnp.where(cnt == 2, np.float32(0.5),
                          jnp.where(cnt == 3, np.float32(1.0 / 3.0),
                                    jnp.where(cnt == 4, np.float32(0.25),
                                              np.float32(0.0)))))
            outv[pl.ds(vo, 16)] = acc * inv

        pltpu.sync_copy(outv, out_hbm.at[pl.ds(j_of(c), BPC)])

    bufs = ((idxv0, gv0, pkv0, semo0, semg0),
            (idxv1, gv1, pkv1, semo1, semg1))

    e0a0, ns0 = meta(jnp.int32(0))
    fire_order_idx(e0a0, ns0, idxv0, semo0)
    fire_packed(jnp.int32(0), pkv0, semo0)
    drain_order(jnp.int32(0), e0a0, ns0, idxv0, pkv0, semo0)
    fire_gath(ns0, idxv0, gv0, semg0)
    e0a1, ns1 = meta(jnp.int32(1))
    fire_order_idx(e0a1, ns1, idxv1, semo1)
    fire_packed(jnp.int32(1), pkv1, semo1)

    def group_body(g, carry):
        for poff in (0, 1):
            c = g * 2 + poff
            idxc, gc, pkc, semoc, semgc = bufs[poff]
            idxn, gn, pkn, semon, semgn = bufs[1 - poff]
            e0a_c, ns_c = meta(c)
            e0a_n, ns_n = meta(c + 1)
            e0a_f, ns_f = meta(c + 2)

            @pl.when(c < ncw - 1)
            def _():
                drain_order(c + 1, e0a_n, ns_n, idxn, pkn, semon)
                fire_gath(ns_n, idxn, gn, semgn)

            drain_gath(ns_c, idxc, gc, semgc)

            @pl.when(c < ncw - 2)
            def _():
                fire_order_idx(e0a_f, ns_f, idxc, semoc)

            compute(c, pkc, gc)

            @pl.when(c < ncw - 2)
            def _():
                fire_packed(c + 2, pkc, semoc)
        return carry

    lax.fori_loop(0, lax.shift_right_logical(ncw, 1), group_body, 0)


_mesh = plsc.VectorSubcoreMesh(core_axis_name="c", subcore_axis_name="s")

_hist = pl.kernel(
    _hist_body, mesh=_mesh,
    compiler_params=pltpu.CompilerParams(needs_layout_passes=False),
    out_type=jax.ShapeDtypeStruct((H * W,), jnp.float32),
    scratch_types=[
        pltpu.VMEM((256,), jnp.int32),
        pltpu.VMEM((LOCCAP,), jnp.int32),
        pltpu.VMEM((LOCCAP,), jnp.int32),
        pltpu.VMEM((LOCCAP,), jnp.float32),
        pltpu.VMEM((LOCCAP,), jnp.float32),
        pltpu.VMEM((BPC,), jnp.int32),
        pltpu.VMEM((BPC,), jnp.int32),
        pltpu.VMEM((BPC,), jnp.float32),
        pltpu.SemaphoreType.DMA,
        pltpu.SemaphoreType.DMA,
        pltpu.SemaphoreType.DMA,
        pltpu.SemaphoreType.DMA,
    ])


def kernel(img):
    order_pad, packed, tbl = _tables()
    P = _hist(img.reshape(-1), order_pad, packed, tbl).reshape(H, W)
    z = img[0, 0]
    col = jnp.arange(W - 1, dtype=jnp.int32)

    def _fix(p):
        c = p[:, : W - 1]
        nz = jnp.any(c != 0.0, axis=0)
        last = jnp.max(jnp.where(nz, col, -1))
        return jnp.where(col[None, :] < last, c, 0.0)

    return lax.cond(z != 0.0, lambda p: p[:, : W - 1], _fix, P)


try:
    _tables()
except Exception:
    pass
